# baseline (speedup 1.0000x reference)
import functools

import jax
import jax.numpy as jnp
from jax import lax
from jax.experimental import pallas as pl
from jax.experimental.pallas import tpu as pltpu
from jax.experimental.pallas import tpu_sc as plsc

VOCAB = 1000000
EMBED = 64
BATCH = 16384
NEG = 10
SLOTS = 16

NUM_CORES = 2
NUM_SUBCORES = 16
LANES = 16
NW = NUM_CORES * NUM_SUBCORES
PER_W = BATCH // NW
CHUNK = 128
NCHUNKS = PER_W // CHUNK


_mesh = plsc.VectorSubcoreMesh(core_axis_name="c", subcore_axis_name="s")


@functools.partial(
    pl.kernel,
    out_type=jax.ShapeDtypeStruct((BATCH, SLOTS), jnp.float32),
    mesh=_mesh,
    compiler_params=pltpu.CompilerParams(needs_layout_passes=False,
                                         use_tc_tiling_on_sc=False),
    scratch_types=[
        pltpu.VMEM((CHUNK,), jnp.int32),
        pltpu.VMEM((CHUNK,), jnp.int32),
        pltpu.VMEM((NEG, CHUNK), jnp.int32),
        pltpu.VMEM((CHUNK, EMBED), jnp.float32),
        pltpu.VMEM((CHUNK, EMBED), jnp.float32),
        pltpu.VMEM((NEG, CHUNK, EMBED), jnp.float32),
        pltpu.VMEM((CHUNK, SLOTS), jnp.float32),
        pltpu.SemaphoreType.DMA,
    ],
)
def _scores_sc(iw_hbm, cw_hbm, negt_hbm, win_hbm, wctx_hbm, out_hbm,
               idx_in, idx_ctx, idx_neg, ei, ec, en, sc_v, sem):
    wid = lax.axis_index("s") * NUM_CORES + lax.axis_index("c")
    lane = lax.iota(jnp.int32, LANES)

    def chunk_body(ci, _):
        base = wid * PER_W + ci * CHUNK
        pltpu.sync_copy(iw_hbm.at[pl.ds(base, CHUNK)], idx_in)
        pltpu.sync_copy(cw_hbm.at[pl.ds(base, CHUNK)], idx_ctx)
        pltpu.sync_copy(negt_hbm.at[:, pl.ds(base, CHUNK)], idx_neg)
        copies = [
            pltpu.async_copy(win_hbm.at[idx_in], ei, sem),
            pltpu.async_copy(wctx_hbm.at[idx_ctx], ec, sem),
        ]
        for k in range(NEG):
            copies.append(pltpu.async_copy(wctx_hbm.at[idx_neg.at[k]],
                                           en.at[k], sem))
        for c in copies:
            c.wait()

        nq = EMBED // LANES

        def j_body(j, _):
            eir = [ei[j, pl.ds(q * LANES, LANES)] for q in range(nq)]
            ecr = [ec[j, pl.ds(q * LANES, LANES)] for q in range(nq)]
            p = eir[0] * ecr[0]
            for q in range(1, nq):
                p = p + eir[q] * ecr[q]
            vals = jnp.where(lane == 0, jnp.sum(p), 0.0)
            for k in range(NEG):
                enr = [en[k, j, pl.ds(q * LANES, LANES)] for q in range(nq)]
                p = eir[0] * enr[0]
                for q in range(1, nq):
                    p = p + eir[q] * enr[q]
                vals = jnp.where(lane == k + 1, -jnp.sum(p), vals)
            sc_v[j, :] = vals
            return 0

        lax.fori_loop(0, CHUNK, j_body, 0)

        pltpu.sync_copy(sc_v, out_hbm.at[pl.ds(base, CHUNK)])
        return 0

    lax.fori_loop(0, NCHUNKS, chunk_body, 0)


def _loss_tc(scores_ref, out_ref):
    x = scores_ref[...]
    col = lax.broadcasted_iota(jnp.int32, x.shape, 1)
    ls = jnp.minimum(x, 0.0) - jnp.log1p(jnp.exp(-jnp.abs(x)))
    m = jnp.where(col < NEG + 1, ls, 0.0)
    out_ref[0, 0] = -jnp.sum(m) / scores_ref.shape[0]


def kernel(input_word, context_word, W_in, W_ctx):
    neg = jax.random.randint(jax.random.key(1234), (BATCH, NEG), 0, VOCAB)
    neg_t = neg.astype(jnp.int32).T
    iw = input_word.astype(jnp.int32)
    cw = context_word.astype(jnp.int32)
    scores = _scores_sc(iw, cw, neg_t, W_in, W_ctx)
    loss = pl.pallas_call(
        _loss_tc,
        out_shape=jax.ShapeDtypeStruct((1, 1), jnp.float32),
        out_specs=pl.BlockSpec(memory_space=pltpu.SMEM),
    )(scores)
    return loss[0, 0]
